# Initial kernel scaffold; baseline (speedup 1.0000x reference)
import jax
import jax.numpy as jnp
from jax.experimental import pallas as pl


def kernel(point1, point2):
    raise NotImplementedError("write your pallas kernel here")


# speedup vs baseline: 17.7949x; 17.7949x over previous
import functools

import jax
import jax.numpy as jnp
from jax.experimental import pallas as pl
from jax.experimental.pallas import tpu as pltpu

_K = 16


def _knn_lap_kernel(p1t_ref, p1T_ref, p1f_ref, p2t_ref, p2f_ref, out_ref,
                    d_ref, *, tile_m, n):
    b = pl.program_id(0)
    i = pl.program_id(1)

    p1T = p1T_ref[0]
    x = p1T[0:1, :]
    y = p1T[1:2, :]
    z = p1T[2:3, :]
    sqj = x * x + y * y + z * z

    p1t = p1t_ref[0]
    tx = p1t[:, 0:1]
    ty = p1t[:, 1:2]
    tz = p1t[:, 2:3]

    d = sqj - 2.0 * (tx * x + ty * y + tz * z)

    rows = jax.lax.broadcasted_iota(jnp.int32, (tile_m, n), 0) + i * tile_m
    cols = jax.lax.broadcasted_iota(jnp.int32, (tile_m, n), 1)
    d = jnp.where(cols == rows, jnp.inf, d)
    d_ref[...] = d

    def body(_, carry):
        dd = d_ref[...]
        m = jnp.min(dd, axis=1, keepdims=True)
        first = jnp.min(jnp.where(dd == m, cols, n), axis=1, keepdims=True)
        d_ref[...] = jnp.where(cols == first, jnp.inf, dd)
        return carry

    jax.lax.fori_loop(0, _K, body, 0)

    adj = jnp.where(jnp.isinf(d_ref[...]) & (cols != rows), 1.0, 0.0)
    s1 = jnp.dot(adj, p1f_ref[0], preferred_element_type=jnp.float32)
    s2 = jnp.dot(adj, p2f_ref[0], preferred_element_type=jnp.float32)
    diff = (s1 - s2) * (1.0 / _K) - (p1t - p2t_ref[0])
    part = jnp.sum(jnp.abs(diff))

    @pl.when((b == 0) & (i == 0))
    def _init():
        out_ref[...] = jnp.zeros_like(out_ref)

    out_ref[...] += part


def kernel(point1, point2):
    B, n, _ = point1.shape
    tile_m = min(256, n)
    p1T = jnp.transpose(point1, (0, 2, 1))
    grid = (B, n // tile_m)
    kern = functools.partial(_knn_lap_kernel, tile_m=tile_m, n=n)
    total = pl.pallas_call(
        kern,
        grid=grid,
        in_specs=[
            pl.BlockSpec((1, tile_m, 3), lambda b, i: (b, i, 0)),
            pl.BlockSpec((1, 3, n), lambda b, i: (b, 0, 0)),
            pl.BlockSpec((1, n, 3), lambda b, i: (b, 0, 0)),
            pl.BlockSpec((1, tile_m, 3), lambda b, i: (b, i, 0)),
            pl.BlockSpec((1, n, 3), lambda b, i: (b, 0, 0)),
        ],
        out_specs=pl.BlockSpec((1, 1), lambda b, i: (0, 0)),
        out_shape=jax.ShapeDtypeStruct((1, 1), jnp.float32),
        scratch_shapes=[pltpu.VMEM((tile_m, n), jnp.float32)],
    )(point1, p1T, point1, point2, point2)
    return total[0, 0] / (B * n * 3)


# speedup vs baseline: 35.9884x; 2.0224x over previous
import functools

import jax
import jax.numpy as jnp
from jax.experimental import pallas as pl
from jax.experimental.pallas import tpu as pltpu

_K = 16


def _knn_lap_kernel(p1t_ref, p1T_ref, p1f_ref, p2t_ref, p2f_ref, out_ref,
                    d_ref, *, tile_m, n):
    b = pl.program_id(0)
    i = pl.program_id(1)

    p1T = p1T_ref[0]
    x = p1T[0:1, :]
    y = p1T[1:2, :]
    z = p1T[2:3, :]
    sqj = x * x + y * y + z * z

    p1t = p1t_ref[0]
    tx = p1t[:, 0:1]
    ty = p1t[:, 1:2]
    tz = p1t[:, 2:3]

    d = sqj - 2.0 * (tx * x + ty * y + tz * z)

    rows = jax.lax.broadcasted_iota(jnp.int32, (tile_m, n), 0) + i * tile_m
    cols = jax.lax.broadcasted_iota(jnp.int32, (tile_m, n), 1)
    d = jnp.where(cols == rows, jnp.inf, d)
    d_ref[...] = d

    for _ in range(_K):
        dd = d_ref[...]
        m = jnp.min(dd, axis=1, keepdims=True)
        d_ref[...] = jnp.where(dd == m, jnp.inf, dd)

    adj = jnp.where(jnp.isinf(d_ref[...]) & (cols != rows), 1.0, 0.0)
    cnt = jnp.sum(adj, axis=1, keepdims=True)
    s1 = jnp.dot(adj, p1f_ref[0], preferred_element_type=jnp.float32)
    s2 = jnp.dot(adj, p2f_ref[0], preferred_element_type=jnp.float32)
    diff = (s1 - s2) / cnt - (p1t - p2t_ref[0])
    part = jnp.sum(jnp.abs(diff))

    @pl.when((b == 0) & (i == 0))
    def _init():
        out_ref[...] = jnp.zeros_like(out_ref)

    out_ref[...] += part


def kernel(point1, point2):
    B, n, _ = point1.shape
    tile_m = min(256, n)
    p1T = jnp.transpose(point1, (0, 2, 1))
    grid = (B, n // tile_m)
    kern = functools.partial(_knn_lap_kernel, tile_m=tile_m, n=n)
    total = pl.pallas_call(
        kern,
        grid=grid,
        in_specs=[
            pl.BlockSpec((1, tile_m, 3), lambda b, i: (b, i, 0)),
            pl.BlockSpec((1, 3, n), lambda b, i: (b, 0, 0)),
            pl.BlockSpec((1, n, 3), lambda b, i: (b, 0, 0)),
            pl.BlockSpec((1, tile_m, 3), lambda b, i: (b, i, 0)),
            pl.BlockSpec((1, n, 3), lambda b, i: (b, 0, 0)),
        ],
        out_specs=pl.BlockSpec((1, 1), lambda b, i: (0, 0)),
        out_shape=jax.ShapeDtypeStruct((1, 1), jnp.float32),
        scratch_shapes=[pltpu.VMEM((tile_m, n), jnp.float32)],
    )(point1, p1T, point1, point2, point2)
    return total[0, 0] / (B * n * 3)
